# baseline (speedup 1.0000x reference)
import jax
import jax.numpy as jnp
from jax import lax
from jax.experimental import pallas as pl
from jax.experimental.pallas import tpu as pltpu

_NH = 4
_NPNT = 4


def _body(x1_ref, x2_ref, qpos_ref, ln1w_ref, ln1b_ref, ln2w_ref, ln2b_ref,
          sow_ref, sob_ref, aww_ref, awb_ref, vpw_ref, vpb_ref,
          opw_ref, opb_ref, out_ref):
    nq, C = x1_ref.shape[1], x1_ref.shape[2]
    W = 32
    hd = C // _NH

    x1b = x1_ref[0]
    x2b = x2_ref[0]

    def ln(x, w, b):
        mu = jnp.mean(x, axis=-1, keepdims=True)
        xc = x - mu
        var = jnp.mean(xc * xc, axis=-1, keepdims=True)
        return xc * lax.rsqrt(var + 1e-5) * w + b

    query = ln(x1b, ln1w_ref[0], ln1b_ref[0]) + qpos_ref[...]
    value = ln(x2b, ln2w_ref[0], ln2b_ref[0])

    v = jnp.dot(value, vpw_ref[...], preferred_element_type=jnp.float32) + vpb_ref[0]

    soT = lax.dot_general(sow_ref[...], query, (((0,), (1,)), ((), ())),
                          preferred_element_type=jnp.float32) + sob_ref[...].reshape(-1, 1)
    awT = lax.dot_general(aww_ref[...], query, (((0,), (1,)), ((), ())),
                          preferred_element_type=jnp.float32) + awb_ref[...].reshape(-1, 1)

    qi = lax.broadcasted_iota(jnp.int32, (1, nq), 1)
    colq = (qi % W).astype(jnp.float32)
    rowq = (qi // W).astype(jnp.float32)
    xg = lax.broadcasted_iota(jnp.int32, (W, nq), 0).astype(jnp.float32)

    outs = []
    for h in range(_NH):
        rows = [awT[h * _NPNT + p:h * _NPNT + p + 1, :] for p in range(_NPNT)]
        m = jnp.maximum(jnp.maximum(rows[0], rows[1]), jnp.maximum(rows[2], rows[3]))
        es = [jnp.exp(r - m) for r in rows]
        denom = es[0] + es[1] + es[2] + es[3]
        inv = 1.0 / denom

        at3 = None
        for p in range(_NPNT):
            o = (h * _NPNT + p) * 2
            x = colq + soT[o:o + 1, :]
            y = rowq + soT[o + 1:o + 2, :]
            wx = jnp.maximum(1.0 - jnp.abs(x - xg), 0.0)
            wy = jnp.maximum(1.0 - jnp.abs(y - xg), 0.0)
            wxa = wx * (es[p] * inv)
            term = wy[:, None, :] * wxa[None, :, :]
            at3 = term if at3 is None else at3 + term
        atm = at3.reshape(nq, nq)
        v_h = v[:, h * hd:(h + 1) * hd]
        out_h = lax.dot_general(atm, v_h, (((0,), (0,)), ((), ())),
                                preferred_element_type=jnp.float32)
        outs.append(out_h)

    sampled = jnp.concatenate(outs, axis=1)
    final = jnp.dot(sampled, opw_ref[...], preferred_element_type=jnp.float32)
    out_ref[0] = final + opb_ref[0] + x2b


def kernel(x1, x2, ln1_w, ln1_b, ln2_w, ln2_b, pos_scale, so_w, so_b,
           aw_w, aw_b, vp_w, vp_b, op_w, op_b):
    B, C, H, W = x1.shape
    nq = H * W

    x1t = x1.reshape(B, C, nq).transpose(0, 2, 1)
    x2t = x2.reshape(B, C, nq).transpose(0, 2, 1)

    inv_freq = 1.0 / (10000.0 ** (jnp.arange(0, C, 2, dtype=jnp.float32) / C))
    t = jnp.arange(nq, dtype=jnp.float32)
    sinu = t[:, None] * inv_freq[None, :]
    qpos = jnp.concatenate([jnp.sin(sinu), jnp.cos(sinu)], axis=-1) * pos_scale

    full = lambda shape: pl.BlockSpec(shape, lambda b: (0,) * len(shape))
    out = pl.pallas_call(
        _body,
        grid=(B,),
        in_specs=[
            pl.BlockSpec((1, nq, C), lambda b: (b, 0, 0)),
            pl.BlockSpec((1, nq, C), lambda b: (b, 0, 0)),
            full((nq, C)),
            full((1, C)), full((1, C)), full((1, C)), full((1, C)),
            full((C, _NH * _NPNT * 2)), full((_NH * _NPNT * 2,)),
            full((C, _NH * _NPNT)), full((_NH * _NPNT,)),
            full((C, C)), full((1, C)),
            full((C, C)), full((1, C)),
        ],
        out_specs=pl.BlockSpec((1, nq, C), lambda b: (b, 0, 0)),
        out_shape=jax.ShapeDtypeStruct((B, nq, C), jnp.float32),
        compiler_params=pltpu.CompilerParams(
            allow_input_fusion=[True, True] + [False] * 13),
    )(x1t, x2t, qpos,
      ln1_w.reshape(1, C), ln1_b.reshape(1, C), ln2_w.reshape(1, C), ln2_b.reshape(1, C),
      so_w, so_b, aw_w, aw_b,
      vp_w, vp_b.reshape(1, C), op_w, op_b.reshape(1, C))
    return out.transpose(0, 2, 1).reshape(B, C, H, W)


# speedup vs baseline: 1.2651x; 1.2651x over previous
import jax
import jax.numpy as jnp
from jax import lax
from jax.experimental import pallas as pl

_NH = 4
_NPNT = 4


def _body(x1_ref, x2_ref, qpos_ref, ln1w_ref, ln1b_ref, ln2w_ref, ln2b_ref,
          sow_ref, sob_ref, aww_ref, awb_ref, vpw_ref, vpb_ref,
          opw_ref, opb_ref, out_ref):
    nq, C = x1_ref.shape[1], x1_ref.shape[2]
    W = 32
    hd = C // _NH

    x1b = x1_ref[0]
    x2b = x2_ref[0]

    def ln(x, w, b):
        mu = jnp.mean(x, axis=-1, keepdims=True)
        xc = x - mu
        var = jnp.mean(xc * xc, axis=-1, keepdims=True)
        return xc * lax.rsqrt(var + 1e-5) * w + b

    query = ln(x1b, ln1w_ref[0], ln1b_ref[0]) + qpos_ref[...]
    value = ln(x2b, ln2w_ref[0], ln2b_ref[0])

    v = jnp.dot(value, vpw_ref[...], preferred_element_type=jnp.float32) + vpb_ref[0]

    soT = lax.dot_general(sow_ref[...], query, (((0,), (1,)), ((), ())),
                          preferred_element_type=jnp.float32) + sob_ref[...].reshape(-1, 1)
    awT = lax.dot_general(aww_ref[...], query, (((0,), (1,)), ((), ())),
                          preferred_element_type=jnp.float32) + awb_ref[...].reshape(-1, 1)

    qi = lax.broadcasted_iota(jnp.int32, (1, nq), 1)
    colq = (qi % W).astype(jnp.float32)
    rowq = (qi // W).astype(jnp.float32)
    xg = lax.broadcasted_iota(jnp.int32, (W, nq), 0).astype(jnp.float32)

    outs = []
    for h in range(_NH):
        rows = [awT[h * _NPNT + p:h * _NPNT + p + 1, :] for p in range(_NPNT)]
        m = jnp.maximum(jnp.maximum(rows[0], rows[1]), jnp.maximum(rows[2], rows[3]))
        es = [jnp.exp(r - m) for r in rows]
        denom = es[0] + es[1] + es[2] + es[3]
        inv = 1.0 / denom

        at3 = None
        for p in range(_NPNT):
            o = (h * _NPNT + p) * 2
            x = colq + soT[o:o + 1, :]
            y = rowq + soT[o + 1:o + 2, :]
            wx = jnp.maximum(1.0 - jnp.abs(x - xg), 0.0)
            wy = jnp.maximum(1.0 - jnp.abs(y - xg), 0.0)
            wxa = (wx * (es[p] * inv)).astype(jnp.bfloat16)
            wyb = wy.astype(jnp.bfloat16)
            term = wyb[:, None, :] * wxa[None, :, :]
            at3 = term if at3 is None else at3 + term
        atm = at3.reshape(nq, nq)
        v_h = v[:, h * hd:(h + 1) * hd].astype(jnp.bfloat16)
        out_h = lax.dot_general(atm, v_h, (((0,), (0,)), ((), ())),
                                preferred_element_type=jnp.float32)
        outs.append(out_h)

    sampled = jnp.concatenate(outs, axis=1)
    final = jnp.dot(sampled, opw_ref[...], preferred_element_type=jnp.float32)
    out_ref[0] = final + opb_ref[0] + x2b


def kernel(x1, x2, ln1_w, ln1_b, ln2_w, ln2_b, pos_scale, so_w, so_b,
           aw_w, aw_b, vp_w, vp_b, op_w, op_b):
    B, C, H, W = x1.shape
    nq = H * W

    x1t = x1.reshape(B, C, nq).transpose(0, 2, 1)
    x2t = x2.reshape(B, C, nq).transpose(0, 2, 1)

    inv_freq = 1.0 / (10000.0 ** (jnp.arange(0, C, 2, dtype=jnp.float32) / C))
    t = jnp.arange(nq, dtype=jnp.float32)
    sinu = t[:, None] * inv_freq[None, :]
    qpos = jnp.concatenate([jnp.sin(sinu), jnp.cos(sinu)], axis=-1) * pos_scale

    full = lambda shape: pl.BlockSpec(shape, lambda b: (0,) * len(shape))
    out = pl.pallas_call(
        _body,
        grid=(B,),
        in_specs=[
            pl.BlockSpec((1, nq, C), lambda b: (b, 0, 0)),
            pl.BlockSpec((1, nq, C), lambda b: (b, 0, 0)),
            full((nq, C)),
            full((1, C)), full((1, C)), full((1, C)), full((1, C)),
            full((C, _NH * _NPNT * 2)), full((_NH * _NPNT * 2,)),
            full((C, _NH * _NPNT)), full((_NH * _NPNT,)),
            full((C, C)), full((1, C)),
            full((C, C)), full((1, C)),
        ],
        out_specs=pl.BlockSpec((1, nq, C), lambda b: (b, 0, 0)),
        out_shape=jax.ShapeDtypeStruct((B, nq, C), jnp.float32),
    )(x1t, x2t, qpos,
      ln1_w.reshape(1, C), ln1_b.reshape(1, C), ln2_w.reshape(1, C), ln2_b.reshape(1, C),
      so_w, so_b, aw_w, aw_b,
      vp_w, vp_b.reshape(1, C), op_w, op_b.reshape(1, C))
    return out.transpose(0, 2, 1).reshape(B, C, H, W)


# speedup vs baseline: 1.2734x; 1.0065x over previous
import jax
import jax.numpy as jnp
from jax import lax
from jax.experimental import pallas as pl

_NH = 4
_NPNT = 4


def _body(x1_ref, x2_ref, soc_ref, awc_ref,
          sow_ref, aww_ref, vpw_ref, vpb_ref,
          opw_ref, opb_ref, out_ref):
    nq, C = x1_ref.shape[1], x1_ref.shape[2]
    W = 32
    hd = C // _NH

    x1b = x1_ref[0]
    x2b = x2_ref[0]

    def ln_core(x):
        mu = jnp.mean(x, axis=-1, keepdims=True)
        xc = x - mu
        var = jnp.mean(xc * xc, axis=-1, keepdims=True)
        return xc * lax.rsqrt(var + 1e-5)

    qn = ln_core(x1b)
    vn = ln_core(x2b)

    v = jnp.dot(vn, vpw_ref[...], preferred_element_type=jnp.float32) + vpb_ref[0]

    soT = lax.dot_general(sow_ref[...], qn, (((0,), (1,)), ((), ())),
                          preferred_element_type=jnp.float32) + soc_ref[...]
    awT = lax.dot_general(aww_ref[...], qn, (((0,), (1,)), ((), ())),
                          preferred_element_type=jnp.float32) + awc_ref[...]

    qi = lax.broadcasted_iota(jnp.int32, (1, nq), 1)
    colq = (qi % W).astype(jnp.float32)
    rowq = (qi // W).astype(jnp.float32)
    xg = lax.broadcasted_iota(jnp.int32, (W, nq), 0).astype(jnp.float32)

    outs = []
    for h in range(_NH):
        rows = [awT[h * _NPNT + p:h * _NPNT + p + 1, :] for p in range(_NPNT)]
        m = jnp.maximum(jnp.maximum(rows[0], rows[1]), jnp.maximum(rows[2], rows[3]))
        es = [jnp.exp(r - m) for r in rows]
        denom = es[0] + es[1] + es[2] + es[3]
        inv = 1.0 / denom

        at3 = None
        for p in range(_NPNT):
            o = (h * _NPNT + p) * 2
            x = colq + soT[o:o + 1, :]
            y = rowq + soT[o + 1:o + 2, :]
            wx = jnp.maximum(1.0 - jnp.abs(x - xg), 0.0)
            wy = jnp.maximum(1.0 - jnp.abs(y - xg), 0.0)
            wxa = (wx * (es[p] * inv)).astype(jnp.bfloat16)
            wyb = wy.astype(jnp.bfloat16)
            term = wyb[:, None, :] * wxa[None, :, :]
            at3 = term if at3 is None else at3 + term
        atm = at3.reshape(nq, nq)
        v_h = v[:, h * hd:(h + 1) * hd].astype(jnp.bfloat16)
        out_h = lax.dot_general(atm, v_h, (((0,), (0,)), ((), ())),
                                preferred_element_type=jnp.float32)
        outs.append(out_h)

    sampled = jnp.concatenate(outs, axis=1)
    final = jnp.dot(sampled, opw_ref[...], preferred_element_type=jnp.float32)
    out_ref[0] = final + opb_ref[0] + x2b


def kernel(x1, x2, ln1_w, ln1_b, ln2_w, ln2_b, pos_scale, so_w, so_b,
           aw_w, aw_b, vp_w, vp_b, op_w, op_b):
    B, C, H, W = x1.shape
    nq = H * W

    x1t = x1.reshape(B, C, nq).transpose(0, 2, 1)
    x2t = x2.reshape(B, C, nq).transpose(0, 2, 1)

    inv_freq = 1.0 / (10000.0 ** (jnp.arange(0, C, 2, dtype=jnp.float32) / C))
    t = jnp.arange(nq, dtype=jnp.float32)
    sinu = t[:, None] * inv_freq[None, :]
    qpos = jnp.concatenate([jnp.sin(sinu), jnp.cos(sinu)], axis=-1) * pos_scale

    qbase = qpos + ln1_b[None, :]
    so_w2 = ln1_w[:, None] * so_w
    aw_w2 = ln1_w[:, None] * aw_w
    soc = (qbase @ so_w + so_b[None, :]).T
    awc = (qbase @ aw_w + aw_b[None, :]).T
    vp_w2 = ln2_w[:, None] * vp_w
    vp_b2 = ln2_b @ vp_w + vp_b

    full = lambda shape: pl.BlockSpec(shape, lambda b: (0,) * len(shape))
    out = pl.pallas_call(
        _body,
        grid=(B,),
        in_specs=[
            pl.BlockSpec((1, nq, C), lambda b: (b, 0, 0)),
            pl.BlockSpec((1, nq, C), lambda b: (b, 0, 0)),
            full((_NH * _NPNT * 2, nq)), full((_NH * _NPNT, nq)),
            full((C, _NH * _NPNT * 2)),
            full((C, _NH * _NPNT)),
            full((C, C)), full((1, C)),
            full((C, C)), full((1, C)),
        ],
        out_specs=pl.BlockSpec((1, nq, C), lambda b: (b, 0, 0)),
        out_shape=jax.ShapeDtypeStruct((B, nq, C), jnp.float32),
    )(x1t, x2t, soc, awc,
      so_w2, aw_w2,
      vp_w2, vp_b2.reshape(1, C), op_w, op_b.reshape(1, C))
    return out.transpose(0, 2, 1).reshape(B, C, H, W)
